# Initial kernel scaffold; baseline (speedup 1.0000x reference)
import jax
import jax.numpy as jnp
from jax.experimental import pallas as pl


def kernel(node_emb, drug_emb, W_gcn, b_gcn, W_neigh, W_self, b_sage, edge_index, bi_edge_index):
    raise NotImplementedError("write your pallas kernel here")


# speedup vs baseline: 8.2604x; 8.2604x over previous
import functools

import jax
import jax.numpy as jnp
from jax import lax
from jax.experimental import pallas as pl
from jax.experimental.pallas import tpu as pltpu
from jax.experimental.pallas import tpu_sc as plsc

N = 10000
D_IN = 128
H = 64
E = 320000

NC = 2
NS = 16
NW = NC * NS
LANE = 128
CH = 80
EPT = CH * LANE
NP = 10240
RPT = NP // NS

_mesh = plsc.VectorSubcoreMesh(core_axis_name="c", subcore_axis_name="s")


@functools.partial(
    pl.kernel,
    out_type=jax.ShapeDtypeStruct((NC * NP,), jnp.float32),
    mesh=_mesh,
    scratch_types=[
        pltpu.VMEM_SHARED((NP,), jnp.float32),
        pltpu.VMEM((CH, LANE), jnp.int32),
        pltpu.VMEM((LANE,), jnp.float32),
        pltpu.VMEM((RPT,), jnp.float32),
    ],
)
def _hist_src(src3, hist_out, h_s, idx_v, ones_v, stg_v):
    cid = lax.axis_index("c")
    sid = lax.axis_index("s")
    wid = cid * NS + sid
    z = jnp.zeros((16,), jnp.float32)
    for i in range(RPT // 16):
        stg_v[pl.ds(i * 16, 16)] = z
    pltpu.sync_copy(stg_v, h_s.at[pl.ds(sid * RPT, RPT)])
    o = jnp.ones((16,), jnp.float32)
    for i in range(LANE // 16):
        ones_v[pl.ds(i * 16, 16)] = o
    plsc.subcore_barrier()
    pltpu.sync_copy(src3.at[wid], idx_v)

    def body(j, carry):
        pltpu.sync_copy(ones_v, h_s.at[idx_v.at[j]], add=True)
        return carry

    lax.fori_loop(0, CH, body, 0)
    plsc.subcore_barrier()
    pltpu.sync_copy(h_s.at[pl.ds(sid * RPT, RPT)], stg_v)
    pltpu.sync_copy(stg_v, hist_out.at[pl.ds(cid * NP + sid * RPT, RPT)])


def _dense_body(ne, de, wg, wn, ws, bs, h0, h1, y, dw, sw):
    deg = h0[...] + h1[...]
    c = jnp.where(deg > 0.0, lax.rsqrt(jnp.maximum(deg, 1.0)), 0.0)
    xw = jnp.dot(ne[...], wg[...], preferred_element_type=jnp.float32)
    y[...] = xw * c
    dw[...] = jnp.dot(de[...], wn[...], preferred_element_type=jnp.float32)
    sw[...] = jnp.dot(ne[...], ws[...], preferred_element_type=jnp.float32) + bs[...]


_dense = pl.pallas_call(
    _dense_body,
    out_shape=(
        jax.ShapeDtypeStruct((N, H), jnp.float32),
        jax.ShapeDtypeStruct((N, H), jnp.float32),
        jax.ShapeDtypeStruct((N, H), jnp.float32),
    ),
)


@functools.partial(
    pl.kernel,
    out_type=(
        jax.ShapeDtypeStruct((NC * NP, H), jnp.float32),
        jax.ShapeDtypeStruct((NC * NP, H), jnp.float32),
        jax.ShapeDtypeStruct((NC * NP,), jnp.float32),
        jax.ShapeDtypeStruct((NC * NP,), jnp.float32),
    ),
    mesh=_mesh,
    compiler_params=pltpu.CompilerParams(use_tc_tiling_on_sc=False),
    scratch_types=[
        pltpu.VMEM_SHARED((NP, H), jnp.float32),
        pltpu.VMEM_SHARED((NP, H), jnp.float32),
        pltpu.VMEM_SHARED((NP,), jnp.float32),
        pltpu.VMEM_SHARED((NP,), jnp.float32),
        pltpu.VMEM((CH, LANE), jnp.int32),
        pltpu.VMEM((CH, LANE), jnp.int32),
        pltpu.VMEM((LANE, H), jnp.float32),
        pltpu.VMEM((LANE,), jnp.float32),
        pltpu.VMEM((16, H), jnp.float32),
        pltpu.VMEM((RPT,), jnp.float32),
        pltpu.SemaphoreType.DMA,
    ],
)
def _edge_passes(y_hbm, dw_hbm, src3, dst3, bsrc3, bdst3,
                 agg1_o, agg2_o, hd_o, hb_o,
                 acc1_s, acc2_s, h1_s, h2_s,
                 sidx_v, didx_v, rows_v, ones_v, zb_v, stg_v, sem):
    cid = lax.axis_index("c")
    sid = lax.axis_index("s")
    wid = cid * NS + sid
    z = jnp.zeros((16,), jnp.float32)
    for r in range(16):
        for cc in range(H // 16):
            zb_v[r, pl.ds(cc * 16, 16)] = z
    for i in range(RPT // 16):
        stg_v[pl.ds(i * 16, 16)] = z
    for k in range(RPT // 16):
        pltpu.sync_copy(zb_v, acc1_s.at[pl.ds(sid * RPT + k * 16, 16)])
        pltpu.sync_copy(zb_v, acc2_s.at[pl.ds(sid * RPT + k * 16, 16)])
    pltpu.sync_copy(stg_v, h1_s.at[pl.ds(sid * RPT, RPT)])
    pltpu.sync_copy(stg_v, h2_s.at[pl.ds(sid * RPT, RPT)])
    o = jnp.ones((16,), jnp.float32)
    for i in range(LANE // 16):
        ones_v[pl.ds(i * 16, 16)] = o
    plsc.subcore_barrier()

    pltpu.sync_copy(src3.at[wid], sidx_v)
    pltpu.sync_copy(dst3.at[wid], didx_v)

    def body1(j, carry):
        pltpu.async_copy(y_hbm.at[sidx_v.at[j]], rows_v, sem).wait()
        pltpu.sync_copy(rows_v, acc1_s.at[didx_v.at[j]], add=True)
        pltpu.sync_copy(ones_v, h1_s.at[didx_v.at[j]], add=True)
        return carry

    lax.fori_loop(0, CH, body1, 0)

    pltpu.sync_copy(bsrc3.at[wid], sidx_v)
    pltpu.sync_copy(bdst3.at[wid], didx_v)

    def body2(j, carry):
        pltpu.async_copy(dw_hbm.at[sidx_v.at[j]], rows_v, sem).wait()
        pltpu.sync_copy(rows_v, acc2_s.at[didx_v.at[j]], add=True)
        pltpu.sync_copy(ones_v, h2_s.at[didx_v.at[j]], add=True)
        return carry

    lax.fori_loop(0, CH, body2, 0)
    plsc.subcore_barrier()

    for k in range(RPT // LANE):
        r0 = sid * RPT + k * LANE
        pltpu.sync_copy(acc1_s.at[pl.ds(r0, LANE)], rows_v)
        pltpu.sync_copy(rows_v, agg1_o.at[pl.ds(cid * NP + r0, LANE)])
        pltpu.sync_copy(acc2_s.at[pl.ds(r0, LANE)], rows_v)
        pltpu.sync_copy(rows_v, agg2_o.at[pl.ds(cid * NP + r0, LANE)])
    pltpu.sync_copy(h1_s.at[pl.ds(sid * RPT, RPT)], stg_v)
    pltpu.sync_copy(stg_v, hd_o.at[pl.ds(cid * NP + sid * RPT, RPT)])
    pltpu.sync_copy(h2_s.at[pl.ds(sid * RPT, RPT)], stg_v)
    pltpu.sync_copy(stg_v, hb_o.at[pl.ds(cid * NP + sid * RPT, RPT)])


def _final_body(a1a, a1b, a2a, a2b, hd0, hd1, hb0, hb1, sw, bg, out):
    agg1 = a1a[...] + a1b[...]
    deg = hd0[...] + hd1[...]
    c = jnp.where(deg > 0.0, lax.rsqrt(jnp.maximum(deg, 1.0)), 0.0)
    gcn = jnp.maximum(agg1 * c + bg[...], 0.0)
    agg2 = a2a[...] + a2b[...]
    cnt = jnp.maximum(hb0[...] + hb1[...], 1.0)
    sage = agg2 / cnt + sw[...]
    out[:, 0:H] = gcn
    out[:, H:2 * H] = sage


_final = pl.pallas_call(
    _final_body,
    out_shape=jax.ShapeDtypeStruct((N, 2 * H), jnp.float32),
)


def _prep_idx(ix, pad_val):
    a = ix.reshape(NW, E // NW)
    p = jnp.full((NW, EPT - E // NW), pad_val, jnp.int32)
    return jnp.concatenate([a, p], axis=1).reshape(NW, CH, LANE)


def kernel(node_emb, drug_emb, W_gcn, b_gcn, W_neigh, W_self, b_sage,
           edge_index, bi_edge_index):
    src = edge_index[0].astype(jnp.int32)
    dst = edge_index[1].astype(jnp.int32)
    bsrc = bi_edge_index[0].astype(jnp.int32)
    bdst = bi_edge_index[1].astype(jnp.int32)

    srcA3 = _prep_idx(src, N)
    srcC3 = _prep_idx(src, 0)
    dst3 = _prep_idx(dst, N)
    bsrc3 = _prep_idx(bsrc, 0)
    bdst3 = _prep_idx(bdst, N)

    hsrc = _hist_src(srcA3)
    h0 = hsrc[0:N].reshape(N, 1)
    h1 = hsrc[NP:NP + N].reshape(N, 1)

    y, dw, sw = _dense(node_emb, drug_emb, W_gcn, W_neigh, W_self,
                       b_sage.reshape(1, H), h0, h1)

    agg1, agg2, hd, hb = _edge_passes(y, dw, srcC3, dst3, bsrc3, bdst3)

    out = _final(agg1[0:N], agg1[NP:NP + N], agg2[0:N], agg2[NP:NP + N],
                 hd[0:N].reshape(N, 1), hd[NP:NP + N].reshape(N, 1),
                 hb[0:N].reshape(N, 1), hb[NP:NP + N].reshape(N, 1),
                 sw, b_gcn.reshape(1, H))
    return out


# speedup vs baseline: 9.7159x; 1.1762x over previous
import functools

import jax
import jax.numpy as jnp
from jax import lax
from jax.experimental import pallas as pl
from jax.experimental.pallas import tpu as pltpu
from jax.experimental.pallas import tpu_sc as plsc

N = 10000
D_IN = 128
H = 64
E = 320000

NC = 2
NS = 16
NW = NC * NS
LANE = 64
CH = 160
EPT = CH * LANE
NP = 10240
RPT = NP // NS

_mesh = plsc.VectorSubcoreMesh(core_axis_name="c", subcore_axis_name="s")


@functools.partial(
    pl.kernel,
    out_type=jax.ShapeDtypeStruct((NC * NP,), jnp.float32),
    mesh=_mesh,
    scratch_types=[
        pltpu.VMEM_SHARED((NP,), jnp.float32),
        pltpu.VMEM((CH, LANE), jnp.int32),
        pltpu.VMEM((LANE,), jnp.float32),
        pltpu.VMEM((RPT,), jnp.float32),
    ],
)
def _hist_src(src3, hist_out, h_s, idx_v, ones_v, stg_v):
    cid = lax.axis_index("c")
    sid = lax.axis_index("s")
    wid = cid * NS + sid
    z = jnp.zeros((16,), jnp.float32)
    for i in range(RPT // 16):
        stg_v[pl.ds(i * 16, 16)] = z
    pltpu.sync_copy(stg_v, h_s.at[pl.ds(sid * RPT, RPT)])
    o = jnp.ones((16,), jnp.float32)
    for i in range(LANE // 16):
        ones_v[pl.ds(i * 16, 16)] = o
    plsc.subcore_barrier()
    pltpu.sync_copy(src3.at[wid], idx_v)

    def body(j, carry):
        pltpu.sync_copy(ones_v, h_s.at[idx_v.at[j]], add=True)
        return carry

    lax.fori_loop(0, CH, body, 0)
    plsc.subcore_barrier()
    pltpu.sync_copy(h_s.at[pl.ds(sid * RPT, RPT)], stg_v)
    pltpu.sync_copy(stg_v, hist_out.at[pl.ds(cid * NP + sid * RPT, RPT)])


def _dense_body(ne, de, wg, wn, ws, bs, h0, h1, y, dw, sw):
    deg = h0[...] + h1[...]
    c = jnp.where(deg > 0.0, lax.rsqrt(jnp.maximum(deg, 1.0)), 0.0)
    xw = jnp.dot(ne[...], wg[...], preferred_element_type=jnp.float32)
    y[...] = xw * c
    dw[...] = jnp.dot(de[...], wn[...], preferred_element_type=jnp.float32)
    sw[...] = jnp.dot(ne[...], ws[...], preferred_element_type=jnp.float32) + bs[...]


_dense = pl.pallas_call(
    _dense_body,
    out_shape=(
        jax.ShapeDtypeStruct((N, H), jnp.float32),
        jax.ShapeDtypeStruct((N, H), jnp.float32),
        jax.ShapeDtypeStruct((N, H), jnp.float32),
    ),
)


NBUF = 4


@functools.partial(
    pl.kernel,
    out_type=(
        jax.ShapeDtypeStruct((NC * NP, H), jnp.float32),
        jax.ShapeDtypeStruct((NC * NP, H), jnp.float32),
        jax.ShapeDtypeStruct((NC * NP,), jnp.float32),
        jax.ShapeDtypeStruct((NC * NP,), jnp.float32),
    ),
    mesh=_mesh,
    compiler_params=pltpu.CompilerParams(use_tc_tiling_on_sc=False),
    scratch_types=[
        pltpu.VMEM_SHARED((NP, H), jnp.float32),
        pltpu.VMEM_SHARED((NP, H), jnp.float32),
        pltpu.VMEM_SHARED((NP,), jnp.float32),
        pltpu.VMEM_SHARED((NP,), jnp.float32),
        pltpu.VMEM((CH, LANE), jnp.int32),
        pltpu.VMEM((CH, LANE), jnp.int32),
        [pltpu.VMEM((LANE, H), jnp.float32)] * NBUF,
        pltpu.VMEM((LANE,), jnp.float32),
        pltpu.VMEM((16, H), jnp.float32),
        pltpu.VMEM((RPT,), jnp.float32),
        [pltpu.SemaphoreType.DMA] * NBUF,
        [pltpu.SemaphoreType.DMA] * NBUF,
        [pltpu.SemaphoreType.DMA] * NBUF,
    ],
)
def _edge_passes(y_hbm, dw_hbm, src3, dst3, bsrc3, bdst3,
                 agg1_o, agg2_o, hd_o, hb_o,
                 acc1_s, acc2_s, h1_s, h2_s,
                 sidx_v, didx_v, rows, ones_v, zb_v, stg_v,
                 gsem, ssem, hsem):
    cid = lax.axis_index("c")
    sid = lax.axis_index("s")
    wid = cid * NS + sid
    z = jnp.zeros((16,), jnp.float32)
    for r in range(16):
        for cc in range(H // 16):
            zb_v[r, pl.ds(cc * 16, 16)] = z
    for i in range(RPT // 16):
        stg_v[pl.ds(i * 16, 16)] = z
    for k in range(RPT // 16):
        pltpu.sync_copy(zb_v, acc1_s.at[pl.ds(sid * RPT + k * 16, 16)])
        pltpu.sync_copy(zb_v, acc2_s.at[pl.ds(sid * RPT + k * 16, 16)])
    pltpu.sync_copy(stg_v, h1_s.at[pl.ds(sid * RPT, RPT)])
    pltpu.sync_copy(stg_v, h2_s.at[pl.ds(sid * RPT, RPT)])
    o = jnp.ones((16,), jnp.float32)
    for i in range(LANE // 16):
        ones_v[pl.ds(i * 16, 16)] = o
    plsc.subcore_barrier()

    def run_pass(tbl_hbm, acc_s, h_s, s3, d3):
        pltpu.sync_copy(s3.at[wid], sidx_v)
        pltpu.sync_copy(d3.at[wid], didx_v)

        def gd(j, b):
            return pltpu.make_async_copy(tbl_hbm.at[sidx_v.at[j]], rows[b], gsem[b])

        def sd(j, b):
            return pltpu.make_async_copy(rows[b], acc_s.at[didx_v.at[j]], ssem[b])

        def hd(j, b):
            return pltpu.make_async_copy(ones_v, h_s.at[didx_v.at[j]], hsem[b])

        for b in range(NBUF):
            gd(b, b).start()

        def outer(g, carry):
            j0 = g * NBUF
            for b in range(NBUF):
                gd(j0 + b, b).wait()
                sd(j0 + b, b).start(add=True)
                hd(j0 + b, b).start(add=True)
            for b in range(NBUF):
                sd(j0 + b, b).wait()
                hd(j0 + b, b).wait()
                gd(j0 + NBUF + b, b).start()
            return carry

        lax.fori_loop(0, CH // NBUF - 1, outer, 0)
        j0 = CH - NBUF
        for b in range(NBUF):
            gd(j0 + b, b).wait()
            sd(j0 + b, b).start(add=True)
            hd(j0 + b, b).start(add=True)
        for b in range(NBUF):
            sd(j0 + b, b).wait()
            hd(j0 + b, b).wait()

    run_pass(y_hbm, acc1_s, h1_s, src3, dst3)
    run_pass(dw_hbm, acc2_s, h2_s, bsrc3, bdst3)
    plsc.subcore_barrier()

    for k in range(RPT // LANE):
        r0 = sid * RPT + k * LANE
        pltpu.sync_copy(acc1_s.at[pl.ds(r0, LANE)], rows[0])
        pltpu.sync_copy(acc2_s.at[pl.ds(r0, LANE)], rows[1])
        pltpu.async_copy(rows[0], agg1_o.at[pl.ds(cid * NP + r0, LANE)], gsem[0]).wait()
        pltpu.async_copy(rows[1], agg2_o.at[pl.ds(cid * NP + r0, LANE)], gsem[1]).wait()
    pltpu.sync_copy(h1_s.at[pl.ds(sid * RPT, RPT)], stg_v)
    pltpu.sync_copy(stg_v, hd_o.at[pl.ds(cid * NP + sid * RPT, RPT)])
    pltpu.sync_copy(h2_s.at[pl.ds(sid * RPT, RPT)], stg_v)
    pltpu.sync_copy(stg_v, hb_o.at[pl.ds(cid * NP + sid * RPT, RPT)])


def _final_body(a1a, a1b, a2a, a2b, hd0, hd1, hb0, hb1, sw, bg, out):
    agg1 = a1a[...] + a1b[...]
    deg = hd0[...] + hd1[...]
    c = jnp.where(deg > 0.0, lax.rsqrt(jnp.maximum(deg, 1.0)), 0.0)
    gcn = jnp.maximum(agg1 * c + bg[...], 0.0)
    agg2 = a2a[...] + a2b[...]
    cnt = jnp.maximum(hb0[...] + hb1[...], 1.0)
    sage = agg2 / cnt + sw[...]
    out[:, 0:H] = gcn
    out[:, H:2 * H] = sage


_final = pl.pallas_call(
    _final_body,
    out_shape=jax.ShapeDtypeStruct((N, 2 * H), jnp.float32),
)


def _prep_idx(ix, pad_val):
    a = ix.reshape(NW, E // NW)
    p = jnp.full((NW, EPT - E // NW), pad_val, jnp.int32)
    return jnp.concatenate([a, p], axis=1).reshape(NW, CH, LANE)


def kernel(node_emb, drug_emb, W_gcn, b_gcn, W_neigh, W_self, b_sage,
           edge_index, bi_edge_index):
    src = edge_index[0].astype(jnp.int32)
    dst = edge_index[1].astype(jnp.int32)
    bsrc = bi_edge_index[0].astype(jnp.int32)
    bdst = bi_edge_index[1].astype(jnp.int32)

    srcA3 = _prep_idx(src, N)
    srcC3 = _prep_idx(src, 0)
    dst3 = _prep_idx(dst, N)
    bsrc3 = _prep_idx(bsrc, 0)
    bdst3 = _prep_idx(bdst, N)

    hsrc = _hist_src(srcA3)
    h0 = hsrc[0:N].reshape(N, 1)
    h1 = hsrc[NP:NP + N].reshape(N, 1)

    y, dw, sw = _dense(node_emb, drug_emb, W_gcn, W_neigh, W_self,
                       b_sage.reshape(1, H), h0, h1)

    agg1, agg2, hd, hb = _edge_passes(y, dw, srcC3, dst3, bsrc3, bdst3)

    out = _final(agg1[0:N], agg1[NP:NP + N], agg2[0:N], agg2[NP:NP + N],
                 hd[0:N].reshape(N, 1), hd[NP:NP + N].reshape(N, 1),
                 hb[0:N].reshape(N, 1), hb[NP:NP + N].reshape(N, 1),
                 sw, b_gcn.reshape(1, H))
    return out


# speedup vs baseline: 10.5370x; 1.0845x over previous
import functools

import jax
import jax.numpy as jnp
from jax import lax
from jax.experimental import pallas as pl
from jax.experimental.pallas import tpu as pltpu
from jax.experimental.pallas import tpu_sc as plsc

N = 10000
D_IN = 128
H = 64
E = 320000

NC = 2
NS = 16
NW = NC * NS
LANE = 128
CH = 160
CHA = 80
NP = 10240
RPT = NP // NS
NBUF = 5

_mesh = plsc.VectorSubcoreMesh(core_axis_name="c", subcore_axis_name="s")


@functools.partial(
    pl.kernel,
    out_type=jax.ShapeDtypeStruct((NC * NP,), jnp.float32),
    mesh=_mesh,
    scratch_types=[
        pltpu.VMEM_SHARED((NP,), jnp.float32),
        pltpu.VMEM((CHA, LANE), jnp.int32),
        pltpu.VMEM((LANE,), jnp.float32),
        pltpu.VMEM((RPT,), jnp.float32),
    ],
)
def _hist_src(src3, hist_out, h_s, idx_v, ones_v, stg_v):
    cid = lax.axis_index("c")
    sid = lax.axis_index("s")
    wid = cid * NS + sid
    z = jnp.zeros((16,), jnp.float32)
    for i in range(RPT // 16):
        stg_v[pl.ds(i * 16, 16)] = z
    pltpu.sync_copy(stg_v, h_s.at[pl.ds(sid * RPT, RPT)])
    o = jnp.ones((16,), jnp.float32)
    for i in range(LANE // 16):
        ones_v[pl.ds(i * 16, 16)] = o
    plsc.subcore_barrier()
    pltpu.sync_copy(src3.at[wid], idx_v)

    def body(j, carry):
        pltpu.sync_copy(ones_v, h_s.at[idx_v.at[j]], add=True)
        return carry

    lax.fori_loop(0, CHA, body, 0)
    plsc.subcore_barrier()
    pltpu.sync_copy(h_s.at[pl.ds(sid * RPT, RPT)], stg_v)
    pltpu.sync_copy(stg_v, hist_out.at[pl.ds(cid * NP + sid * RPT, RPT)])


def _dense_body(ne, de, wg, wn, ws, bs, h0, h1, tbl, sw):
    deg = h0[...] + h1[...]
    c = jnp.where(deg > 0.0, lax.rsqrt(jnp.maximum(deg, 1.0)), 0.0)
    xw = jnp.dot(ne[...], wg[...], preferred_element_type=jnp.float32)
    tbl[0:N, :] = xw * c
    tbl[N:2 * N, :] = jnp.dot(de[...], wn[...], preferred_element_type=jnp.float32)
    sw[...] = jnp.dot(ne[...], ws[...], preferred_element_type=jnp.float32) + bs[...]


_dense = pl.pallas_call(
    _dense_body,
    out_shape=(
        jax.ShapeDtypeStruct((2 * N, H), jnp.float32),
        jax.ShapeDtypeStruct((N, H), jnp.float32),
    ),
)


@functools.partial(
    pl.kernel,
    out_type=(
        jax.ShapeDtypeStruct((NC * NP, H), jnp.float32),
        jax.ShapeDtypeStruct((NC * NP,), jnp.float32),
    ),
    mesh=_mesh,
    compiler_params=pltpu.CompilerParams(use_tc_tiling_on_sc=False),
    scratch_types=[
        pltpu.VMEM_SHARED((NP, H), jnp.float32),
        pltpu.VMEM_SHARED((NP,), jnp.float32),
        pltpu.VMEM((CH, LANE), jnp.int32),
        pltpu.VMEM((CH, LANE), jnp.int32),
        [pltpu.VMEM((LANE, H), jnp.float32)] * NBUF,
        pltpu.VMEM((LANE,), jnp.float32),
        pltpu.VMEM((16, H), jnp.float32),
        pltpu.VMEM((RPT,), jnp.float32),
        [pltpu.SemaphoreType.DMA] * NBUF,
        [pltpu.SemaphoreType.DMA] * NBUF,
        [pltpu.SemaphoreType.DMA] * NBUF,
    ],
)
def _edge_passes(tbl_hbm, esrc3, edst3,
                 agg_o, hist_o,
                 acc_s, h_s,
                 sidx_v, didx_v, rows, ones_v, zb_v, stg_v,
                 gsem, ssem, hsem):
    cid = lax.axis_index("c")
    sid = lax.axis_index("s")
    wid = cid * NS + sid
    z = jnp.zeros((16,), jnp.float32)
    for r in range(16):
        for cc in range(H // 16):
            zb_v[r, pl.ds(cc * 16, 16)] = z
    for i in range(RPT // 16):
        stg_v[pl.ds(i * 16, 16)] = z
    for k in range(RPT // 16):
        pltpu.sync_copy(zb_v, acc_s.at[pl.ds(sid * RPT + k * 16, 16)])
    pltpu.sync_copy(stg_v, h_s.at[pl.ds(sid * RPT, RPT)])
    o = jnp.ones((16,), jnp.float32)
    for i in range(LANE // 16):
        ones_v[pl.ds(i * 16, 16)] = o
    plsc.subcore_barrier()

    pltpu.sync_copy(esrc3.at[wid], sidx_v)
    pltpu.sync_copy(edst3.at[wid], didx_v)

    def gd(j, b):
        return pltpu.make_async_copy(tbl_hbm.at[sidx_v.at[j]], rows[b], gsem[b])

    def sd(j, b):
        return pltpu.make_async_copy(rows[b], acc_s.at[didx_v.at[j]], ssem[b])

    def hd(j, b):
        return pltpu.make_async_copy(ones_v, h_s.at[didx_v.at[j]], hsem[b])

    for b in range(NBUF):
        gd(b, b).start()

    def outer(g, carry):
        j0 = g * NBUF
        for b in range(NBUF):
            gd(j0 + b, b).wait()
            sd(j0 + b, b).start(add=True)
            hd(j0 + b, b).start(add=True)
        for b in range(NBUF):
            sd(j0 + b, b).wait()
            hd(j0 + b, b).wait()
            gd(j0 + NBUF + b, b).start()
        return carry

    lax.fori_loop(0, CH // NBUF - 1, outer, 0)
    j0 = CH - NBUF
    for b in range(NBUF):
        gd(j0 + b, b).wait()
        sd(j0 + b, b).start(add=True)
        hd(j0 + b, b).start(add=True)
    for b in range(NBUF):
        sd(j0 + b, b).wait()
        hd(j0 + b, b).wait()
    plsc.subcore_barrier()

    for k in range(RPT // LANE):
        r0 = sid * RPT + k * LANE
        pltpu.sync_copy(acc_s.at[pl.ds(r0, LANE)], rows[0])
        pltpu.async_copy(rows[0], agg_o.at[pl.ds(cid * NP + r0, LANE)], gsem[0]).wait()
    pltpu.sync_copy(h_s.at[pl.ds(sid * RPT, RPT)], stg_v)
    pltpu.sync_copy(stg_v, hist_o.at[pl.ds(cid * NP + sid * RPT, RPT)])


def _final_body(a1, a2, hd, hb, sw, bg, out):
    deg = hd[...]
    c = jnp.where(deg > 0.0, lax.rsqrt(jnp.maximum(deg, 1.0)), 0.0)
    gcn = jnp.maximum(a1[...] * c + bg[...], 0.0)
    cnt = jnp.maximum(hb[...], 1.0)
    sage = a2[...] / cnt + sw[...]
    out[:, 0:H] = gcn
    out[:, H:2 * H] = sage


_final = pl.pallas_call(
    _final_body,
    out_shape=jax.ShapeDtypeStruct((N, 2 * H), jnp.float32),
)


def _prep_hist_idx(ix, pad_val):
    a = ix.reshape(NW, E // NW)
    p = jnp.full((NW, CHA * LANE - E // NW), pad_val, jnp.int32)
    return jnp.concatenate([a, p], axis=1).reshape(NW, CHA, LANE)


def _prep_edge_idx(ix1, ix2, pad1, pad2):
    a1 = ix1.reshape(NS, E // NS)
    a2 = ix2.reshape(NS, E // NS)
    p1 = jnp.full((NS, CH * LANE - E // NS), pad1, jnp.int32)
    p2 = jnp.full((NS, CH * LANE - E // NS), pad2, jnp.int32)
    g1 = jnp.concatenate([a1, p1], axis=1)
    g2 = jnp.concatenate([a2, p2], axis=1)
    return jnp.concatenate([g1, g2], axis=0).reshape(NW, CH, LANE)


def kernel(node_emb, drug_emb, W_gcn, b_gcn, W_neigh, W_self, b_sage,
           edge_index, bi_edge_index):
    src = edge_index[0].astype(jnp.int32)
    dst = edge_index[1].astype(jnp.int32)
    bsrc = bi_edge_index[0].astype(jnp.int32)
    bdst = bi_edge_index[1].astype(jnp.int32)

    srcA3 = _prep_hist_idx(src, N)
    esrc3 = _prep_edge_idx(src, bsrc + N, 0, 0)
    edst3 = _prep_edge_idx(dst, bdst, N, N)

    hsrc = _hist_src(srcA3)
    h0 = hsrc[0:N].reshape(N, 1)
    h1 = hsrc[NP:NP + N].reshape(N, 1)

    tbl, sw = _dense(node_emb, drug_emb, W_gcn, W_neigh, W_self,
                     b_sage.reshape(1, H), h0, h1)

    agg, hist = _edge_passes(tbl, esrc3, edst3)

    out = _final(agg[0:N], agg[NP:NP + N],
                 hist[0:N].reshape(N, 1), hist[NP:NP + N].reshape(N, 1),
                 sw, b_gcn.reshape(1, H))
    return out


# speedup vs baseline: 20.1242x; 1.9099x over previous
import functools

import jax
import jax.numpy as jnp
from jax import lax
from jax.experimental import pallas as pl
from jax.experimental.pallas import tpu as pltpu
from jax.experimental.pallas import tpu_sc as plsc

N = 10000
D_IN = 128
H = 64
E = 320000

NC = 2
NS = 16
NW = NC * NS
LANE = 128
CH = 160
CHA = 80
NP = 10240
RPT = NP // NS
NBUF = 4
IDXD = 8
TPT = N // NS

_mesh = plsc.VectorSubcoreMesh(core_axis_name="c", subcore_axis_name="s")


@functools.partial(
    pl.kernel,
    out_type=jax.ShapeDtypeStruct((NC * NP,), jnp.float32),
    mesh=_mesh,
    scratch_types=[
        pltpu.VMEM_SHARED((NP,), jnp.float32),
        pltpu.VMEM((CHA, LANE), jnp.int32),
        pltpu.VMEM((LANE,), jnp.float32),
        pltpu.VMEM((RPT,), jnp.float32),
    ],
)
def _hist_src(src3, hist_out, h_s, idx_v, ones_v, stg_v):
    cid = lax.axis_index("c")
    sid = lax.axis_index("s")
    wid = cid * NS + sid
    z = jnp.zeros((16,), jnp.float32)
    for i in range(RPT // 16):
        stg_v[pl.ds(i * 16, 16)] = z
    pltpu.sync_copy(stg_v, h_s.at[pl.ds(sid * RPT, RPT)])
    o = jnp.ones((16,), jnp.float32)
    for i in range(LANE // 16):
        ones_v[pl.ds(i * 16, 16)] = o
    plsc.subcore_barrier()
    pltpu.sync_copy(src3.at[wid], idx_v)

    def body(j, carry):
        pltpu.sync_copy(ones_v, h_s.at[idx_v.at[j]], add=True)
        return carry

    lax.fori_loop(0, CHA, body, 0)
    plsc.subcore_barrier()
    pltpu.sync_copy(h_s.at[pl.ds(sid * RPT, RPT)], stg_v)
    pltpu.sync_copy(stg_v, hist_out.at[pl.ds(cid * NP + sid * RPT, RPT)])


def _dense_body(ne, de, wg, wn, ws, bs, h0, h1, tbl, sw):
    deg = h0[...] + h1[...]
    c = jnp.where(deg > 0.0, lax.rsqrt(jnp.maximum(deg, 1.0)), 0.0)
    xw = jnp.dot(ne[...], wg[...], preferred_element_type=jnp.float32)
    tbl[0:N, :] = xw * c
    tbl[N:2 * N, :] = jnp.dot(de[...], wn[...], preferred_element_type=jnp.float32)
    sw[...] = jnp.dot(ne[...], ws[...], preferred_element_type=jnp.float32) + bs[...]


_dense = pl.pallas_call(
    _dense_body,
    out_shape=(
        jax.ShapeDtypeStruct((2 * N, H), jnp.float32),
        jax.ShapeDtypeStruct((N, H), jnp.float32),
    ),
)


@functools.partial(
    pl.kernel,
    out_type=(
        jax.ShapeDtypeStruct((NC * NP, H), jnp.float32),
        jax.ShapeDtypeStruct((NC * NP,), jnp.float32),
    ),
    mesh=_mesh,
    compiler_params=pltpu.CompilerParams(use_tc_tiling_on_sc=False),
    scratch_types=[
        pltpu.VMEM_SHARED((N, H), jnp.float32),
        pltpu.VMEM_SHARED((NP, H), jnp.float32),
        pltpu.VMEM_SHARED((NP,), jnp.float32),
        pltpu.VMEM((IDXD, LANE), jnp.int32),
        pltpu.VMEM((IDXD, LANE), jnp.int32),
        [pltpu.VMEM((LANE, H), jnp.float32)] * NBUF,
        pltpu.VMEM((LANE,), jnp.float32),
        pltpu.VMEM((16, H), jnp.float32),
        pltpu.VMEM((RPT,), jnp.float32),
        [pltpu.SemaphoreType.DMA] * NBUF,
        [pltpu.SemaphoreType.DMA] * NBUF,
        [pltpu.SemaphoreType.DMA] * NBUF,
        [pltpu.SemaphoreType.DMA] * IDXD,
    ],
)
def _edge_passes(tbl_hbm, esrc2, edst2,
                 agg_o, hist_o,
                 tbl_s, acc_s, h_s,
                 sidxr, didxr, rows, ones_v, zb_v, stg_v,
                 gsem, ssem, hsem, isem):
    cid = lax.axis_index("c")
    sid = lax.axis_index("s")
    wid = cid * NS + sid
    z = jnp.zeros((16,), jnp.float32)
    for r in range(16):
        for cc in range(H // 16):
            zb_v[r, pl.ds(cc * 16, 16)] = z
    for i in range(RPT // 16):
        stg_v[pl.ds(i * 16, 16)] = z
    for k in range(RPT // 16):
        pltpu.sync_copy(zb_v, acc_s.at[pl.ds(sid * RPT + k * 16, 16)])
    pltpu.sync_copy(stg_v, h_s.at[pl.ds(sid * RPT, RPT)])
    o = jnp.ones((16,), jnp.float32)
    for i in range(LANE // 16):
        ones_v[pl.ds(i * 16, 16)] = o
    pltpu.sync_copy(tbl_hbm.at[pl.ds(cid * N + sid * TPT, TPT)],
                    tbl_s.at[pl.ds(sid * TPT, TPT)])
    plsc.subcore_barrier()

    cbase = wid * CH

    def il(j, s):
        return (pltpu.make_async_copy(esrc2.at[cbase + j], sidxr.at[s], isem[s]),
                pltpu.make_async_copy(edst2.at[cbase + j], didxr.at[s], isem[s]))

    def gd(s, b):
        return pltpu.make_async_copy(tbl_s.at[sidxr.at[s]], rows[b], gsem[b])

    def sd(s, b):
        return pltpu.make_async_copy(rows[b], acc_s.at[didxr.at[s]], ssem[b])

    def hd(s, b):
        return pltpu.make_async_copy(ones_v, h_s.at[didxr.at[s]], hsem[b])

    def il_start(j, s):
        a, d = il(j, s)
        a.start()
        d.start()

    def il_wait(j, s):
        a, d = il(j, s)
        a.wait()
        d.wait()

    for b in range(4):
        il_start(b, b)
    for b in range(4):
        il_start(4 + b, 4 + b)
    for b in range(4):
        il_wait(b, b)
        gd(b, b).start()

    def pair_body(p, carry):
        j0 = 8 * p
        for b in range(4):
            gd(b, b).wait()
            sd(b, b).start(add=True)
            hd(b, b).start(add=True)
        for b in range(4):
            sd(b, b).wait()
            hd(b, b).wait()
            il_start(j0 + 8 + b, b)
        for b in range(4):
            il_wait(j0 + 4 + b, 4 + b)
            gd(4 + b, b).start()
        for b in range(4):
            gd(4 + b, b).wait()
            sd(4 + b, b).start(add=True)
            hd(4 + b, b).start(add=True)
        for b in range(4):
            sd(4 + b, b).wait()
            hd(4 + b, b).wait()
            il_start(j0 + 12 + b, 4 + b)
        for b in range(4):
            il_wait(j0 + 8 + b, b)
            gd(b, b).start()
        return carry

    lax.fori_loop(0, (CH // 8) - 1, pair_body, 0)

    for b in range(4):
        gd(b, b).wait()
        sd(b, b).start(add=True)
        hd(b, b).start(add=True)
    for b in range(4):
        sd(b, b).wait()
        hd(b, b).wait()
    for b in range(4):
        il_wait(156 + b, 4 + b)
        gd(4 + b, b).start()
    for b in range(4):
        gd(4 + b, b).wait()
        sd(4 + b, b).start(add=True)
        hd(4 + b, b).start(add=True)
    for b in range(4):
        sd(4 + b, b).wait()
        hd(4 + b, b).wait()
    plsc.subcore_barrier()

    for k in range(RPT // LANE):
        r0 = sid * RPT + k * LANE
        pltpu.sync_copy(acc_s.at[pl.ds(r0, LANE)], rows[0])
        pltpu.async_copy(rows[0], agg_o.at[pl.ds(cid * NP + r0, LANE)], gsem[0]).wait()
    pltpu.sync_copy(h_s.at[pl.ds(sid * RPT, RPT)], stg_v)
    pltpu.sync_copy(stg_v, hist_o.at[pl.ds(cid * NP + sid * RPT, RPT)])


def _final_body(a1, a2, hd, hb, sw, bg, out):
    deg = hd[...]
    c = jnp.where(deg > 0.0, lax.rsqrt(jnp.maximum(deg, 1.0)), 0.0)
    gcn = jnp.maximum(a1[...] * c + bg[...], 0.0)
    cnt = jnp.maximum(hb[...], 1.0)
    sage = a2[...] / cnt + sw[...]
    out[:, 0:H] = gcn
    out[:, H:2 * H] = sage


_final = pl.pallas_call(
    _final_body,
    out_shape=jax.ShapeDtypeStruct((N, 2 * H), jnp.float32),
)


def _prep_hist_idx(ix, pad_val):
    a = ix.reshape(NW, E // NW)
    p = jnp.full((NW, CHA * LANE - E // NW), pad_val, jnp.int32)
    return jnp.concatenate([a, p], axis=1).reshape(NW, CHA, LANE)


def _prep_edge_idx(ix1, ix2, pad1, pad2):
    a1 = ix1.reshape(NS, E // NS)
    a2 = ix2.reshape(NS, E // NS)
    p1 = jnp.full((NS, CH * LANE - E // NS), pad1, jnp.int32)
    p2 = jnp.full((NS, CH * LANE - E // NS), pad2, jnp.int32)
    g1 = jnp.concatenate([a1, p1], axis=1)
    g2 = jnp.concatenate([a2, p2], axis=1)
    return jnp.concatenate([g1, g2], axis=0).reshape(NW * CH, LANE)


def kernel(node_emb, drug_emb, W_gcn, b_gcn, W_neigh, W_self, b_sage,
           edge_index, bi_edge_index):
    src = edge_index[0].astype(jnp.int32)
    dst = edge_index[1].astype(jnp.int32)
    bsrc = bi_edge_index[0].astype(jnp.int32)
    bdst = bi_edge_index[1].astype(jnp.int32)

    srcA3 = _prep_hist_idx(src, N)
    esrc2 = _prep_edge_idx(src, bsrc, 0, 0)
    edst2 = _prep_edge_idx(dst, bdst, N, N)

    hsrc = _hist_src(srcA3)
    h0 = hsrc[0:N].reshape(N, 1)
    h1 = hsrc[NP:NP + N].reshape(N, 1)

    tbl, sw = _dense(node_emb, drug_emb, W_gcn, W_neigh, W_self,
                     b_sage.reshape(1, H), h0, h1)

    agg, hist = _edge_passes(tbl, esrc2, edst2)

    out = _final(agg[0:N], agg[NP:NP + N],
                 hist[0:N].reshape(N, 1), hist[NP:NP + N].reshape(N, 1),
                 sw, b_gcn.reshape(1, H))
    return out


# speedup vs baseline: 20.1254x; 1.0001x over previous
import functools

import jax
import jax.numpy as jnp
from jax import lax
from jax.experimental import pallas as pl
from jax.experimental.pallas import tpu as pltpu
from jax.experimental.pallas import tpu_sc as plsc

N = 10000
D_IN = 128
H = 64
E = 320000

NC = 2
NS = 16
NW = NC * NS
LANE = 128
CH = 160
CHA = 80
NP = 10240
RPT = NP // NS
NBUF = 4
IDXD = 8
TPT = N // NS

_mesh = plsc.VectorSubcoreMesh(core_axis_name="c", subcore_axis_name="s")


@functools.partial(
    pl.kernel,
    out_type=jax.ShapeDtypeStruct((NC * NP,), jnp.float32),
    mesh=_mesh,
    scratch_types=[
        pltpu.VMEM_SHARED((NP,), jnp.float32),
        pltpu.VMEM((CHA, LANE), jnp.int32),
        pltpu.VMEM((LANE,), jnp.float32),
        pltpu.VMEM((RPT,), jnp.float32),
        [pltpu.SemaphoreType.DMA] * 4,
    ],
)
def _hist_src(src3, hist_out, h_s, idx_v, ones_v, stg_v, asem):
    cid = lax.axis_index("c")
    sid = lax.axis_index("s")
    wid = cid * NS + sid
    z = jnp.zeros((16,), jnp.float32)
    for i in range(RPT // 16):
        stg_v[pl.ds(i * 16, 16)] = z
    pltpu.sync_copy(stg_v, h_s.at[pl.ds(sid * RPT, RPT)])
    o = jnp.ones((16,), jnp.float32)
    for i in range(LANE // 16):
        ones_v[pl.ds(i * 16, 16)] = o
    plsc.subcore_barrier()
    pltpu.sync_copy(src3.at[wid], idx_v)

    def hd(j, b):
        return pltpu.make_async_copy(ones_v, h_s.at[idx_v.at[j]], asem[b])

    for b in range(4):
        hd(b, b).start(add=True)

    def body(g, carry):
        j0 = 4 * g
        for b in range(4):
            hd(j0 + b, b).wait()
            hd(j0 + 4 + b, b).start(add=True)
        return carry

    lax.fori_loop(0, CHA // 4 - 1, body, 0)
    for b in range(4):
        hd(CHA - 4 + b, b).wait()
    plsc.subcore_barrier()
    pltpu.sync_copy(h_s.at[pl.ds(sid * RPT, RPT)], stg_v)
    pltpu.sync_copy(stg_v, hist_out.at[pl.ds(cid * NP + sid * RPT, RPT)])


def _dense_body(ne, de, wg, wn, ws, bs, h0, h1, tbl, sw):
    deg = h0[...] + h1[...]
    c = jnp.where(deg > 0.0, lax.rsqrt(jnp.maximum(deg, 1.0)), 0.0)
    xw = jnp.dot(ne[...], wg[...], preferred_element_type=jnp.float32)
    tbl[0:N, :] = xw * c
    tbl[N:2 * N, :] = jnp.dot(de[...], wn[...], preferred_element_type=jnp.float32)
    sw[...] = jnp.dot(ne[...], ws[...], preferred_element_type=jnp.float32) + bs[...]


_dense = pl.pallas_call(
    _dense_body,
    out_shape=(
        jax.ShapeDtypeStruct((2 * N, H), jnp.float32),
        jax.ShapeDtypeStruct((N, H), jnp.float32),
    ),
)


@functools.partial(
    pl.kernel,
    out_type=(
        jax.ShapeDtypeStruct((NC * NP, H), jnp.float32),
        jax.ShapeDtypeStruct((NC * NP,), jnp.float32),
    ),
    mesh=_mesh,
    compiler_params=pltpu.CompilerParams(use_tc_tiling_on_sc=False),
    scratch_types=[
        pltpu.VMEM_SHARED((N, H), jnp.float32),
        pltpu.VMEM_SHARED((NP, H), jnp.float32),
        pltpu.VMEM_SHARED((NP,), jnp.float32),
        pltpu.VMEM((IDXD, LANE), jnp.int32),
        pltpu.VMEM((IDXD, LANE), jnp.int32),
        [pltpu.VMEM((LANE, H), jnp.float32)] * NBUF,
        pltpu.VMEM((LANE,), jnp.float32),
        pltpu.VMEM((16, H), jnp.float32),
        pltpu.VMEM((RPT,), jnp.float32),
        [pltpu.SemaphoreType.DMA] * NBUF,
        [pltpu.SemaphoreType.DMA] * NBUF,
        [pltpu.SemaphoreType.DMA] * NBUF,
        [pltpu.SemaphoreType.DMA] * IDXD,
    ],
)
def _edge_passes(tbl_hbm, esrc2, edst2,
                 agg_o, hist_o,
                 tbl_s, acc_s, h_s,
                 sidxr, didxr, rows, ones_v, zb_v, stg_v,
                 gsem, ssem, hsem, isem):
    cid = lax.axis_index("c")
    sid = lax.axis_index("s")
    wid = cid * NS + sid
    z = jnp.zeros((16,), jnp.float32)
    for r in range(16):
        for cc in range(H // 16):
            zb_v[r, pl.ds(cc * 16, 16)] = z
    for i in range(RPT // 16):
        stg_v[pl.ds(i * 16, 16)] = z
    for k in range(RPT // 16):
        pltpu.sync_copy(zb_v, acc_s.at[pl.ds(sid * RPT + k * 16, 16)])
    pltpu.sync_copy(stg_v, h_s.at[pl.ds(sid * RPT, RPT)])
    o = jnp.ones((16,), jnp.float32)
    for i in range(LANE // 16):
        ones_v[pl.ds(i * 16, 16)] = o
    pltpu.sync_copy(tbl_hbm.at[pl.ds(cid * N + sid * TPT, TPT)],
                    tbl_s.at[pl.ds(sid * TPT, TPT)])
    plsc.subcore_barrier()

    cbase = wid * CH

    def il(j, s):
        return (pltpu.make_async_copy(esrc2.at[cbase + j], sidxr.at[s], isem[s]),
                pltpu.make_async_copy(edst2.at[cbase + j], didxr.at[s], isem[s]))

    def gd(s, b):
        return pltpu.make_async_copy(tbl_s.at[sidxr.at[s]], rows[b], gsem[b])

    def sd(s, b):
        return pltpu.make_async_copy(rows[b], acc_s.at[didxr.at[s]], ssem[b])

    def hd(s, b):
        return pltpu.make_async_copy(ones_v, h_s.at[didxr.at[s]], hsem[b])

    def il_start(j, s):
        a, d = il(j, s)
        a.start()
        d.start()

    def il_wait(j, s):
        a, d = il(j, s)
        a.wait()
        d.wait()

    for b in range(4):
        il_start(b, b)
    for b in range(4):
        il_start(4 + b, 4 + b)
    for b in range(4):
        il_wait(b, b)
        gd(b, b).start()

    def pair_body(p, carry):
        j0 = 8 * p
        for b in range(4):
            gd(b, b).wait()
            sd(b, b).start(add=True)
            hd(b, b).start(add=True)
        for b in range(4):
            sd(b, b).wait()
            hd(b, b).wait()
            il_start(j0 + 8 + b, b)
        for b in range(4):
            il_wait(j0 + 4 + b, 4 + b)
            gd(4 + b, b).start()
        for b in range(4):
            gd(4 + b, b).wait()
            sd(4 + b, b).start(add=True)
            hd(4 + b, b).start(add=True)
        for b in range(4):
            sd(4 + b, b).wait()
            hd(4 + b, b).wait()
            il_start(j0 + 12 + b, 4 + b)
        for b in range(4):
            il_wait(j0 + 8 + b, b)
            gd(b, b).start()
        return carry

    lax.fori_loop(0, (CH // 8) - 1, pair_body, 0)

    for b in range(4):
        gd(b, b).wait()
        sd(b, b).start(add=True)
        hd(b, b).start(add=True)
    for b in range(4):
        sd(b, b).wait()
        hd(b, b).wait()
    for b in range(4):
        il_wait(156 + b, 4 + b)
        gd(4 + b, b).start()
    for b in range(4):
        gd(4 + b, b).wait()
        sd(4 + b, b).start(add=True)
        hd(4 + b, b).start(add=True)
    for b in range(4):
        sd(4 + b, b).wait()
        hd(4 + b, b).wait()
    plsc.subcore_barrier()

    for k in range(RPT // LANE):
        r0 = sid * RPT + k * LANE
        pltpu.sync_copy(acc_s.at[pl.ds(r0, LANE)], rows[0])
        pltpu.async_copy(rows[0], agg_o.at[pl.ds(cid * NP + r0, LANE)], gsem[0]).wait()
    pltpu.sync_copy(h_s.at[pl.ds(sid * RPT, RPT)], stg_v)
    pltpu.sync_copy(stg_v, hist_o.at[pl.ds(cid * NP + sid * RPT, RPT)])


def _final_body(a1, a2, hd, hb, sw, bg, out):
    deg = hd[...]
    c = jnp.where(deg > 0.0, lax.rsqrt(jnp.maximum(deg, 1.0)), 0.0)
    gcn = jnp.maximum(a1[...] * c + bg[...], 0.0)
    cnt = jnp.maximum(hb[...], 1.0)
    sage = a2[...] / cnt + sw[...]
    out[:, 0:H] = gcn
    out[:, H:2 * H] = sage


_final = pl.pallas_call(
    _final_body,
    out_shape=jax.ShapeDtypeStruct((N, 2 * H), jnp.float32),
)


def _prep_hist_idx(ix, pad_val):
    a = ix.reshape(NW, E // NW)
    p = jnp.full((NW, CHA * LANE - E // NW), pad_val, jnp.int32)
    return jnp.concatenate([a, p], axis=1).reshape(NW, CHA, LANE)


def _prep_edge_idx(ix1, ix2, pad1, pad2):
    a1 = ix1.reshape(NS, E // NS)
    a2 = ix2.reshape(NS, E // NS)
    p1 = jnp.full((NS, CH * LANE - E // NS), pad1, jnp.int32)
    p2 = jnp.full((NS, CH * LANE - E // NS), pad2, jnp.int32)
    g1 = jnp.concatenate([a1, p1], axis=1)
    g2 = jnp.concatenate([a2, p2], axis=1)
    return jnp.concatenate([g1, g2], axis=0).reshape(NW * CH, LANE)


def kernel(node_emb, drug_emb, W_gcn, b_gcn, W_neigh, W_self, b_sage,
           edge_index, bi_edge_index):
    src = edge_index[0].astype(jnp.int32)
    dst = edge_index[1].astype(jnp.int32)
    bsrc = bi_edge_index[0].astype(jnp.int32)
    bdst = bi_edge_index[1].astype(jnp.int32)

    srcA3 = _prep_hist_idx(src, N)
    esrc2 = _prep_edge_idx(src, bsrc, 0, 0)
    edst2 = _prep_edge_idx(dst, bdst, N, N)

    hsrc = _hist_src(srcA3)
    h0 = hsrc[0:N].reshape(N, 1)
    h1 = hsrc[NP:NP + N].reshape(N, 1)

    tbl, sw = _dense(node_emb, drug_emb, W_gcn, W_neigh, W_self,
                     b_sage.reshape(1, H), h0, h1)

    agg, hist = _edge_passes(tbl, esrc2, edst2)

    out = _final(agg[0:N], agg[NP:NP + N],
                 hist[0:N].reshape(N, 1), hist[NP:NP + N].reshape(N, 1),
                 sw, b_gcn.reshape(1, H))
    return out
